# baseline (speedup 1.0000x reference)
import jax
import jax.numpy as jnp
from jax.experimental import pallas as pl
from jax.experimental.pallas import tpu as pltpu

_N_TOK = 65536
_D_IN = 96
_D_LAT = 32
_N_CB = 8192
_HALF = _N_CB // 2
_TB = 256
_EB = 2048


def _enc_body(x_ref, w_ref, b_ref, ze_ref):
    ze_ref[...] = jnp.dot(x_ref[...], w_ref[...]) + b_ref[...]


def _vq_body(ze_ref, s_ref, c_ref, decw_ref, decb_ref, cb_ref,
             recons_ref, zq_ref, idx_ref, losssum_ref):
    z_e = ze_ref[...]
    cb = cb_ref[...]

    dot = jax.lax.dot_general(z_e, cb, (((1,), (1,)), ((), ())))
    dist = (s_ref[...] + c_ref[...]) - 2.0 * dot

    d1 = dist[:, :_HALF]
    d2 = dist[:, _HALF:]
    m1 = jnp.min(d1, axis=1)
    m2 = jnp.min(d2, axis=1)
    iota = jax.lax.broadcasted_iota(jnp.int32, (_TB, _HALF), 1)
    big = jnp.int32(2**30)
    i1 = jnp.min(jnp.where(d1 == m1[:, None], iota, big), axis=1)
    i2 = jnp.min(jnp.where(d2 == m2[:, None], iota, big), axis=1) + _HALF
    q1 = m1.astype(jnp.bfloat16).astype(jnp.float32)
    idx = jnp.where(m2 < q1, i2, i1)

    onehot = (jax.lax.broadcasted_iota(jnp.int32, (_TB, _N_CB), 1)
              == idx[:, None]).astype(jnp.float32)
    z_q = jax.lax.dot_general(onehot, cb, (((1,), (0,)), ((), ())),
                              precision=jax.lax.Precision.HIGHEST)

    recons_ref[...] = jnp.dot(z_q, decw_ref[...]) + decb_ref[...]
    zq_ref[...] = z_q
    idx_ref[...] = idx[:, None]

    diff = z_q - z_e
    bsum = jnp.sum(diff * diff).reshape(1, 1)

    @pl.when(pl.program_id(0) == 0)
    def _init():
        losssum_ref[...] = jnp.zeros((1, 1), jnp.float32)

    losssum_ref[...] += bsum


def kernel(x, enc_W, enc_b, dec_W, dec_b, codebook):
    z_e = pl.pallas_call(
        _enc_body,
        grid=(_N_TOK // _EB,),
        in_specs=[
            pl.BlockSpec((_EB, _D_IN), lambda i: (i, 0)),
            pl.BlockSpec((_D_IN, _D_LAT), lambda i: (0, 0)),
            pl.BlockSpec((1, _D_LAT), lambda i: (0, 0)),
        ],
        out_specs=pl.BlockSpec((_EB, _D_LAT), lambda i: (i, 0)),
        out_shape=jax.ShapeDtypeStruct((_N_TOK, _D_LAT), jnp.float32),
    )(x, enc_W, enc_b.reshape(1, _D_LAT))

    S = jnp.sum(z_e * z_e, axis=1).reshape(_N_TOK, 1)
    c = jnp.sum(codebook * codebook, axis=1).reshape(1, _N_CB)

    recons, z_q, idx2d, losssum = pl.pallas_call(
        _vq_body,
        grid=(_N_TOK // _TB,),
        in_specs=[
            pl.BlockSpec((_TB, _D_LAT), lambda i: (i, 0)),
            pl.BlockSpec((_TB, 1), lambda i: (i, 0)),
            pl.BlockSpec((1, _N_CB), lambda i: (0, 0)),
            pl.BlockSpec((_D_LAT, _D_IN), lambda i: (0, 0)),
            pl.BlockSpec((1, _D_IN), lambda i: (0, 0)),
            pl.BlockSpec((_N_CB, _D_LAT), lambda i: (0, 0)),
        ],
        out_specs=[
            pl.BlockSpec((_TB, _D_IN), lambda i: (i, 0)),
            pl.BlockSpec((_TB, _D_LAT), lambda i: (i, 0)),
            pl.BlockSpec((_TB, 1), lambda i: (i, 0)),
            pl.BlockSpec((1, 1), lambda i: (0, 0)),
        ],
        out_shape=[
            jax.ShapeDtypeStruct((_N_TOK, _D_IN), jnp.float32),
            jax.ShapeDtypeStruct((_N_TOK, _D_LAT), jnp.float32),
            jax.ShapeDtypeStruct((_N_TOK, 1), jnp.int32),
            jax.ShapeDtypeStruct((1, 1), jnp.float32),
        ],
        compiler_params=pltpu.CompilerParams(
            dimension_semantics=("arbitrary",),
        ),
    )(z_e, S, c, dec_W, dec_b.reshape(1, _D_IN), codebook)

    loss = losssum[0, 0] / (_N_TOK * _D_LAT)
    encode_indices = idx2d.reshape(_N_TOK)
    return (recons, z_q, loss, loss, encode_indices)


# speedup vs baseline: 3.1492x; 3.1492x over previous
import functools

import jax
import jax.numpy as jnp
from jax import lax
from jax.experimental import pallas as pl
from jax.experimental.pallas import tpu as pltpu
from jax.experimental.pallas import tpu_sc as plsc

_N_TOK = 65536
_D_IN = 96
_D_LAT = 32
_N_CB = 8192
_HALF = _N_CB // 2
_TB = 256
_EB = 2048
_NBLK = _N_TOK // _TB


def _enc_body(x_ref, w_ref, b_ref, ze_ref):
    ze_ref[...] = jnp.dot(x_ref[...], w_ref[...]) + b_ref[...]


def _vq_body(ze_ref, s_ref, c_ref, cb_ref, idx_ref, lsum_ref):
    z_e = ze_ref[...]
    cb = cb_ref[...]

    dot2 = jax.lax.dot_general(z_e + z_e, cb,
                               (((1,), (1,)), ((), ())))
    dist = (s_ref[...] + c_ref[...]) - dot2

    d1 = dist[:, :_HALF]
    d2 = dist[:, _HALF:]
    m1 = jnp.min(d1, axis=1)
    m2 = jnp.min(d2, axis=1)
    iota = jax.lax.broadcasted_iota(jnp.int32, (_TB, _HALF), 1)
    big = jnp.int32(2**30)
    i1 = jnp.min(jnp.where(d1 == m1[:, None], iota, big), axis=1)
    i2 = jnp.min(jnp.where(d2 == m2[:, None], iota, big), axis=1) + _HALF
    q1 = m1.astype(jnp.bfloat16).astype(jnp.float32)
    take2 = m2 < q1
    idx_ref[...] = jnp.where(take2, i2, i1)[:, None]
    msel = jnp.where(take2, m2, m1)
    lsum_ref[...] = jnp.sum(msel).reshape(1, 1, 1)


def _dec_body(cb_ref, w_ref, b_ref, part_ref, tab_ref, lsum_ref):
    cb = cb_ref[...]
    dec = jnp.dot(cb, w_ref[...]) + b_ref[...]
    tab_ref[...] = jnp.concatenate([cb, dec], axis=1)
    lsum_ref[...] = jnp.sum(part_ref[...]).reshape(1, 1)


def _make_gather():
    info = plsc.get_sparse_core_info()
    nw = info.num_cores * info.num_subcores
    b_per_w = _N_TOK // nw
    ch = 512
    n_ch = b_per_w // ch
    mesh = plsc.VectorSubcoreMesh(core_axis_name="c", subcore_axis_name="s")

    @functools.partial(
        pl.kernel, mesh=mesh,
        out_type=jax.ShapeDtypeStruct((_N_TOK, _D_LAT + _D_IN), jnp.float32),
        scratch_types=[
            pltpu.VMEM((ch,), jnp.int32),
            pltpu.VMEM((ch, _D_LAT + _D_IN), jnp.float32),
            pltpu.SemaphoreType.DMA,
        ],
        name="vq_gather",
    )
    def gather(tab_hbm, idx_hbm, out_hbm, idx_v, rows_v, sem):
        wid = lax.axis_index("s") * info.num_cores + lax.axis_index("c")
        base = wid * b_per_w
        for k in range(n_ch):
            off = base + k * ch
            pltpu.sync_copy(idx_hbm.at[pl.ds(off, ch)], idx_v)
            pltpu.async_copy(tab_hbm.at[idx_v], rows_v, sem).wait()
            pltpu.sync_copy(rows_v, out_hbm.at[pl.ds(off, ch)])

    return gather


_gather = _make_gather()


def kernel(x, enc_W, enc_b, dec_W, dec_b, codebook):
    z_e = pl.pallas_call(
        _enc_body,
        grid=(_N_TOK // _EB,),
        in_specs=[
            pl.BlockSpec((_EB, _D_IN), lambda i: (i, 0)),
            pl.BlockSpec((_D_IN, _D_LAT), lambda i: (0, 0)),
            pl.BlockSpec((1, _D_LAT), lambda i: (0, 0)),
        ],
        out_specs=pl.BlockSpec((_EB, _D_LAT), lambda i: (i, 0)),
        out_shape=jax.ShapeDtypeStruct((_N_TOK, _D_LAT), jnp.float32),
        compiler_params=pltpu.CompilerParams(
            dimension_semantics=("parallel",),
        ),
    )(x, enc_W, enc_b.reshape(1, _D_LAT))

    S = jnp.sum(z_e * z_e, axis=1).reshape(_N_TOK, 1)
    c = jnp.sum(codebook * codebook, axis=1).reshape(1, _N_CB)

    idx2d, lparts = pl.pallas_call(
        _vq_body,
        grid=(_NBLK,),
        in_specs=[
            pl.BlockSpec((_TB, _D_LAT), lambda i: (i, 0)),
            pl.BlockSpec((_TB, 1), lambda i: (i, 0)),
            pl.BlockSpec((1, _N_CB), lambda i: (0, 0)),
            pl.BlockSpec((_N_CB, _D_LAT), lambda i: (0, 0)),
        ],
        out_specs=[
            pl.BlockSpec((_TB, 1), lambda i: (i, 0)),
            pl.BlockSpec((1, 1, 1), lambda i: (i, 0, 0)),
        ],
        out_shape=[
            jax.ShapeDtypeStruct((_N_TOK, 1), jnp.int32),
            jax.ShapeDtypeStruct((_NBLK, 1, 1), jnp.float32),
        ],
        compiler_params=pltpu.CompilerParams(
            dimension_semantics=("parallel",),
        ),
    )(z_e, S, c, codebook)

    table, lsum = pl.pallas_call(
        _dec_body,
        grid=(1,),
        in_specs=[
            pl.BlockSpec((_N_CB, _D_LAT), lambda i: (0, 0)),
            pl.BlockSpec((_D_LAT, _D_IN), lambda i: (0, 0)),
            pl.BlockSpec((1, _D_IN), lambda i: (0, 0)),
            pl.BlockSpec((_NBLK, 1, 1), lambda i: (0, 0, 0)),
        ],
        out_specs=[
            pl.BlockSpec((_N_CB, _D_LAT + _D_IN), lambda i: (0, 0)),
            pl.BlockSpec((1, 1), lambda i: (0, 0)),
        ],
        out_shape=[
            jax.ShapeDtypeStruct((_N_CB, _D_LAT + _D_IN), jnp.float32),
            jax.ShapeDtypeStruct((1, 1), jnp.float32),
        ],
    )(codebook, dec_W, dec_b.reshape(1, _D_IN), lparts)

    encode_indices = idx2d.reshape(_N_TOK)
    rows = _gather(table, encode_indices)
    z_q = rows[:, :_D_LAT]
    recons = rows[:, _D_LAT:]

    loss = lsum[0, 0] / (_N_TOK * _D_LAT)
    return (recons, z_q, loss, loss, encode_indices)


# speedup vs baseline: 3.3277x; 1.0567x over previous
import functools

import jax
import jax.numpy as jnp
from jax import lax
from jax.experimental import pallas as pl
from jax.experimental.pallas import tpu as pltpu
from jax.experimental.pallas import tpu_sc as plsc

_N_TOK = 65536
_D_IN = 96
_D_LAT = 32
_N_CB = 8192
_HALF = _N_CB // 2
_TB = 256
_EB = 2048
_NBLK = _N_TOK // _TB


def _enc_body(x_ref, w_ref, b_ref, ze_ref):
    ze_ref[...] = jnp.dot(x_ref[...], w_ref[...]) + b_ref[...]


def _vq_body(ze_ref, s_ref, c_ref, cb_ref, idx_ref, lsum_ref):
    z_e = ze_ref[...]
    cb = cb_ref[...]

    dot2 = jax.lax.dot_general(z_e + z_e, cb,
                               (((1,), (1,)), ((), ())))
    dist = (s_ref[...] + c_ref[...]) - dot2

    d1 = dist[:, :_HALF]
    d2 = dist[:, _HALF:]
    m1 = jnp.min(d1, axis=1)
    m2 = jnp.min(d2, axis=1)
    iota = jax.lax.broadcasted_iota(jnp.int32, (_TB, _HALF), 1).astype(jnp.float32)
    big = jnp.float32(3e38)
    i1 = jnp.min(jnp.where(d1 == m1[:, None], iota, big), axis=1).astype(jnp.int32)
    i2 = jnp.min(jnp.where(d2 == m2[:, None], iota, big), axis=1).astype(jnp.int32) + _HALF
    q1 = m1.astype(jnp.bfloat16).astype(jnp.float32)
    take2 = m2 < q1
    idx_ref[...] = jnp.where(take2, i2, i1)[:, None]
    msel = jnp.where(take2, m2, m1)
    lsum_ref[...] = jnp.sum(msel).reshape(1, 1, 1)


def _dec_body(cb_ref, w_ref, b_ref, part_ref, tab_ref, lsum_ref):
    cb = cb_ref[...]
    dec = jnp.dot(cb, w_ref[...]) + b_ref[...]
    tab_ref[...] = jnp.concatenate([cb, dec], axis=1)
    lsum_ref[...] = jnp.sum(part_ref[...]).reshape(1, 1)


def _make_gather():
    info = plsc.get_sparse_core_info()
    nw = info.num_cores * info.num_subcores
    b_per_w = _N_TOK // nw
    ch = 512
    n_ch = b_per_w // ch
    mesh = plsc.VectorSubcoreMesh(core_axis_name="c", subcore_axis_name="s")

    @functools.partial(
        pl.kernel, mesh=mesh,
        out_type=jax.ShapeDtypeStruct((_N_TOK, _D_LAT + _D_IN), jnp.float32),
        scratch_types=[
            pltpu.VMEM((ch,), jnp.int32),
            pltpu.VMEM((ch, _D_LAT + _D_IN), jnp.float32),
            pltpu.SemaphoreType.DMA,
        ],
        name="vq_gather",
    )
    def gather(tab_hbm, idx_hbm, out_hbm, idx_v, rows_v, sem):
        wid = lax.axis_index("s") * info.num_cores + lax.axis_index("c")
        base = wid * b_per_w
        for k in range(n_ch):
            off = base + k * ch
            pltpu.sync_copy(idx_hbm.at[pl.ds(off, ch)], idx_v)
            pltpu.async_copy(tab_hbm.at[idx_v], rows_v, sem).wait()
            pltpu.sync_copy(rows_v, out_hbm.at[pl.ds(off, ch)])

    return gather


_gather = _make_gather()


def kernel(x, enc_W, enc_b, dec_W, dec_b, codebook):
    z_e = pl.pallas_call(
        _enc_body,
        grid=(_N_TOK // _EB,),
        in_specs=[
            pl.BlockSpec((_EB, _D_IN), lambda i: (i, 0)),
            pl.BlockSpec((_D_IN, _D_LAT), lambda i: (0, 0)),
            pl.BlockSpec((1, _D_LAT), lambda i: (0, 0)),
        ],
        out_specs=pl.BlockSpec((_EB, _D_LAT), lambda i: (i, 0)),
        out_shape=jax.ShapeDtypeStruct((_N_TOK, _D_LAT), jnp.float32),
        compiler_params=pltpu.CompilerParams(
            dimension_semantics=("parallel",),
        ),
    )(x, enc_W, enc_b.reshape(1, _D_LAT))

    S = jnp.sum(z_e * z_e, axis=1).reshape(_N_TOK, 1)
    c = jnp.sum(codebook * codebook, axis=1).reshape(1, _N_CB)

    idx2d, lparts = pl.pallas_call(
        _vq_body,
        grid=(_NBLK,),
        in_specs=[
            pl.BlockSpec((_TB, _D_LAT), lambda i: (i, 0)),
            pl.BlockSpec((_TB, 1), lambda i: (i, 0)),
            pl.BlockSpec((1, _N_CB), lambda i: (0, 0)),
            pl.BlockSpec((_N_CB, _D_LAT), lambda i: (0, 0)),
        ],
        out_specs=[
            pl.BlockSpec((_TB, 1), lambda i: (i, 0)),
            pl.BlockSpec((1, 1, 1), lambda i: (i, 0, 0)),
        ],
        out_shape=[
            jax.ShapeDtypeStruct((_N_TOK, 1), jnp.int32),
            jax.ShapeDtypeStruct((_NBLK, 1, 1), jnp.float32),
        ],
        compiler_params=pltpu.CompilerParams(
            dimension_semantics=("parallel",),
        ),
    )(z_e, S, c, codebook)

    table, lsum = pl.pallas_call(
        _dec_body,
        grid=(1,),
        in_specs=[
            pl.BlockSpec((_N_CB, _D_LAT), lambda i: (0, 0)),
            pl.BlockSpec((_D_LAT, _D_IN), lambda i: (0, 0)),
            pl.BlockSpec((1, _D_IN), lambda i: (0, 0)),
            pl.BlockSpec((_NBLK, 1, 1), lambda i: (0, 0, 0)),
        ],
        out_specs=[
            pl.BlockSpec((_N_CB, _D_LAT + _D_IN), lambda i: (0, 0)),
            pl.BlockSpec((1, 1), lambda i: (0, 0)),
        ],
        out_shape=[
            jax.ShapeDtypeStruct((_N_CB, _D_LAT + _D_IN), jnp.float32),
            jax.ShapeDtypeStruct((1, 1), jnp.float32),
        ],
    )(codebook, dec_W, dec_b.reshape(1, _D_IN), lparts)

    encode_indices = idx2d.reshape(_N_TOK)
    rows = _gather(table, encode_indices)
    z_q = rows[:, :_D_LAT]
    recons = rows[:, _D_LAT:]

    loss = lsum[0, 0] / (_N_TOK * _D_LAT)
    return (recons, z_q, loss, loss, encode_indices)


# speedup vs baseline: 3.4856x; 1.0475x over previous
import functools

import jax
import jax.numpy as jnp
from jax import lax
from jax.experimental import pallas as pl
from jax.experimental.pallas import tpu as pltpu
from jax.experimental.pallas import tpu_sc as plsc

_N_TOK = 65536
_D_IN = 96
_D_LAT = 32
_N_CB = 8192
_HALF = _N_CB // 2
_TB = 512
_EB = 2048
_NBLK = _N_TOK // _TB


def _enc_body(x_ref, w_ref, b_ref, ze_ref):
    ze_ref[...] = jnp.dot(x_ref[...], w_ref[...]) + b_ref[...]


def _vq_body(ze_ref, s_ref, c_ref, cb_ref, idx_ref, lsum_ref):
    z_e = ze_ref[...]
    cb = cb_ref[...]

    dot2 = jax.lax.dot_general(z_e + z_e, cb,
                               (((1,), (1,)), ((), ())))
    dist = (s_ref[...] + c_ref[...]) - dot2

    d1 = dist[:, :_HALF]
    d2 = dist[:, _HALF:]
    m1 = jnp.min(d1, axis=1)
    m2 = jnp.min(d2, axis=1)
    iota = jax.lax.broadcasted_iota(jnp.int32, (_TB, _HALF), 1).astype(jnp.float32)
    big = jnp.float32(3e38)
    i1 = jnp.min(jnp.where(d1 == m1[:, None], iota, big), axis=1).astype(jnp.int32)
    i2 = jnp.min(jnp.where(d2 == m2[:, None], iota, big), axis=1).astype(jnp.int32) + _HALF
    q1 = m1.astype(jnp.bfloat16).astype(jnp.float32)
    take2 = m2 < q1
    idx_ref[...] = jnp.where(take2, i2, i1)[:, None]
    msel = jnp.where(take2, m2, m1)
    lsum_ref[...] = jnp.sum(msel).reshape(1, 1, 1)


def _dec_body(cb_ref, w_ref, b_ref, part_ref, tab_ref, lsum_ref):
    cb = cb_ref[...]
    dec = jnp.dot(cb, w_ref[...]) + b_ref[...]
    tab_ref[...] = jnp.concatenate([cb, dec], axis=1)
    lsum_ref[...] = jnp.sum(part_ref[...]).reshape(1, 1)


def _make_gather():
    info = plsc.get_sparse_core_info()
    nw = info.num_cores * info.num_subcores
    b_per_w = _N_TOK // nw
    ch = 512
    n_ch = b_per_w // ch
    mesh = plsc.VectorSubcoreMesh(core_axis_name="c", subcore_axis_name="s")

    @functools.partial(
        pl.kernel, mesh=mesh,
        out_type=jax.ShapeDtypeStruct((_N_TOK, _D_LAT + _D_IN), jnp.float32),
        scratch_types=[
            pltpu.VMEM((ch,), jnp.int32),
            pltpu.VMEM((ch, _D_LAT + _D_IN), jnp.float32),
            pltpu.SemaphoreType.DMA,
        ],
        name="vq_gather",
    )
    def gather(tab_hbm, idx_hbm, out_hbm, idx_v, rows_v, sem):
        wid = lax.axis_index("s") * info.num_cores + lax.axis_index("c")
        base = wid * b_per_w
        for k in range(n_ch):
            off = base + k * ch
            pltpu.sync_copy(idx_hbm.at[pl.ds(off, ch)], idx_v)
            pltpu.async_copy(tab_hbm.at[idx_v], rows_v, sem).wait()
            pltpu.sync_copy(rows_v, out_hbm.at[pl.ds(off, ch)])

    return gather


_gather = _make_gather()


def kernel(x, enc_W, enc_b, dec_W, dec_b, codebook):
    z_e = pl.pallas_call(
        _enc_body,
        grid=(_N_TOK // _EB,),
        in_specs=[
            pl.BlockSpec((_EB, _D_IN), lambda i: (i, 0)),
            pl.BlockSpec((_D_IN, _D_LAT), lambda i: (0, 0)),
            pl.BlockSpec((1, _D_LAT), lambda i: (0, 0)),
        ],
        out_specs=pl.BlockSpec((_EB, _D_LAT), lambda i: (i, 0)),
        out_shape=jax.ShapeDtypeStruct((_N_TOK, _D_LAT), jnp.float32),
        compiler_params=pltpu.CompilerParams(
            dimension_semantics=("parallel",),
        ),
    )(x, enc_W, enc_b.reshape(1, _D_LAT))

    S = jnp.sum(z_e * z_e, axis=1).reshape(_N_TOK, 1)
    c = jnp.sum(codebook * codebook, axis=1).reshape(1, _N_CB)

    idx2d, lparts = pl.pallas_call(
        _vq_body,
        grid=(_NBLK,),
        in_specs=[
            pl.BlockSpec((_TB, _D_LAT), lambda i: (i, 0)),
            pl.BlockSpec((_TB, 1), lambda i: (i, 0)),
            pl.BlockSpec((1, _N_CB), lambda i: (0, 0)),
            pl.BlockSpec((_N_CB, _D_LAT), lambda i: (0, 0)),
        ],
        out_specs=[
            pl.BlockSpec((_TB, 1), lambda i: (i, 0)),
            pl.BlockSpec((1, 1, 1), lambda i: (i, 0, 0)),
        ],
        out_shape=[
            jax.ShapeDtypeStruct((_N_TOK, 1), jnp.int32),
            jax.ShapeDtypeStruct((_NBLK, 1, 1), jnp.float32),
        ],
        compiler_params=pltpu.CompilerParams(
            dimension_semantics=("parallel",),
        ),
    )(z_e, S, c, codebook)

    table, lsum = pl.pallas_call(
        _dec_body,
        grid=(1,),
        in_specs=[
            pl.BlockSpec((_N_CB, _D_LAT), lambda i: (0, 0)),
            pl.BlockSpec((_D_LAT, _D_IN), lambda i: (0, 0)),
            pl.BlockSpec((1, _D_IN), lambda i: (0, 0)),
            pl.BlockSpec((_NBLK, 1, 1), lambda i: (0, 0, 0)),
        ],
        out_specs=[
            pl.BlockSpec((_N_CB, _D_LAT + _D_IN), lambda i: (0, 0)),
            pl.BlockSpec((1, 1), lambda i: (0, 0)),
        ],
        out_shape=[
            jax.ShapeDtypeStruct((_N_CB, _D_LAT + _D_IN), jnp.float32),
            jax.ShapeDtypeStruct((1, 1), jnp.float32),
        ],
    )(codebook, dec_W, dec_b.reshape(1, _D_IN), lparts)

    encode_indices = idx2d.reshape(_N_TOK)
    rows = _gather(table, encode_indices)
    z_q = rows[:, :_D_LAT]
    recons = rows[:, _D_LAT:]

    loss = lsum[0, 0] / (_N_TOK * _D_LAT)
    return (recons, z_q, loss, loss, encode_indices)
